# Initial kernel scaffold; baseline (speedup 1.0000x reference)
import jax
import jax.numpy as jnp
from jax.experimental import pallas as pl


def kernel(confidence, predicted_locations, labels, gt_locations):
    raise NotImplementedError("write your pallas kernel here")


# speedup vs baseline: 1.8530x; 1.8530x over previous
import functools

import jax
import jax.numpy as jnp
from jax.experimental import pallas as pl

_NEG_POS_RATIO = 3
_INTERPRET = False


def _phase1_body(conf_ref, labels_ref, pred_ref, gt_ref, bg_ref, cep_ref, sl1_ref):
    x = conf_ref[...]
    m = jnp.max(x, axis=2, keepdims=True)
    lse = m[:, :, 0] + jnp.log(jnp.sum(jnp.exp(x - m), axis=2))
    bg_ref[0] = lse - x[:, :, 0]
    lab = labels_ref[0]
    iota = jax.lax.broadcasted_iota(jnp.int32, x.shape, 2)
    xsel = jnp.sum(jnp.where(iota == lab[:, :, None], x, 0.0), axis=2)
    pos = lab > 0
    cep_ref[0] = jnp.where(pos, lse - xsel, 0.0)
    d = pred_ref[0] - gt_ref[0]
    ad = jnp.abs(d)
    sl1 = jnp.where(ad < 1.0, 0.5 * d * d, ad - 0.5)
    sl1_ref[0] = jnp.where(pos, jnp.sum(sl1, axis=1), 0.0)


def _phase2_body(neg_pos_ratio, n, bg_ref, cep_ref, sl1_ref, labels_ref, o1_ref, o2_ref):
    lab = labels_ref[...]
    pos = lab > 0
    num_pos = jnp.sum(pos.astype(jnp.int32), axis=(0, 2), keepdims=True)
    k = num_pos * neg_pos_ratio
    negcount = n - num_pos
    bg = bg_ref[...]
    bits = jax.lax.bitcast_convert_type(bg, jnp.uint32)
    key = jnp.where(bg >= 0, bits | jnp.uint32(0x80000000), ~bits)
    key = jnp.where(pos, jnp.uint32(0), key)
    sum_neg = jnp.sum(jnp.where(pos, 0.0, bg), axis=(0, 2), keepdims=True)
    prefix = jnp.zeros_like(num_pos, dtype=jnp.uint32)
    kk = k
    for bit in range(31, -1, -1):
        cand = prefix | jnp.uint32(1 << bit)
        match = (key >> jnp.uint32(bit)) == (cand >> jnp.uint32(bit))
        cnt = jnp.sum(match.astype(jnp.int32), axis=(0, 2), keepdims=True)
        take = cnt >= kk
        prefix = jnp.where(take, cand, prefix)
        kk = jnp.where(take, kk, kk - cnt)
    t = prefix
    gt_mask = key > t
    num_gt = jnp.sum(gt_mask.astype(jnp.int32), axis=(0, 2), keepdims=True)
    sum_gt = jnp.sum(jnp.where(gt_mask, bg, 0.0), axis=(0, 2), keepdims=True)
    vt = jnp.max(jnp.where(key == t, bg, -jnp.inf), axis=(0, 2), keepdims=True)
    topk = sum_gt + (k - num_gt).astype(jnp.float32) * vt
    topk = jnp.where(k >= negcount, sum_neg, topk)
    topk = jnp.where(k <= 0, 0.0, topk)
    npos_tot = jnp.sum(num_pos, axis=1, keepdims=True).astype(jnp.float32) + 1e-6
    sl1_tot = jnp.sum(sl1_ref[...], axis=(0, 2), keepdims=True)
    cep_tot = jnp.sum(cep_ref[...], axis=(0, 2), keepdims=True)
    o1_ref[...] = (jnp.sum(sl1_tot, axis=1, keepdims=True) / npos_tot)[0]
    cls = jnp.sum(cep_tot + topk, axis=1, keepdims=True)
    o2_ref[...] = (cls / npos_tot)[0]


def kernel(confidence, predicted_locations, labels, gt_locations):
    b, n, c = confidence.shape
    tn = 400
    nt = n // tn
    labels = labels.astype(jnp.int32)
    labels_t = labels.reshape(b, nt, tn).transpose(1, 0, 2)
    pred_t = predicted_locations.reshape(b, nt, tn, 4).transpose(1, 0, 3, 2)
    gt_t = gt_locations.reshape(b, nt, tn, 4).transpose(1, 0, 3, 2)
    bg, cep, sl1 = pl.pallas_call(
        _phase1_body,
        grid=(nt,),
        in_specs=[
            pl.BlockSpec((b, tn, c), lambda j: (0, j, 0)),
            pl.BlockSpec((1, b, tn), lambda j: (j, 0, 0)),
            pl.BlockSpec((1, b, 4, tn), lambda j: (j, 0, 0, 0)),
            pl.BlockSpec((1, b, 4, tn), lambda j: (j, 0, 0, 0)),
        ],
        out_specs=[
            pl.BlockSpec((1, b, tn), lambda j: (j, 0, 0)),
            pl.BlockSpec((1, b, tn), lambda j: (j, 0, 0)),
            pl.BlockSpec((1, b, tn), lambda j: (j, 0, 0)),
        ],
        out_shape=[
            jax.ShapeDtypeStruct((nt, b, tn), jnp.float32),
            jax.ShapeDtypeStruct((nt, b, tn), jnp.float32),
            jax.ShapeDtypeStruct((nt, b, tn), jnp.float32),
        ],
        interpret=_INTERPRET,
    )(confidence, labels_t, pred_t, gt_t)

    o1, o2 = pl.pallas_call(
        functools.partial(_phase2_body, _NEG_POS_RATIO, n),
        out_shape=[
            jax.ShapeDtypeStruct((1, 1), jnp.float32),
            jax.ShapeDtypeStruct((1, 1), jnp.float32),
        ],
        interpret=_INTERPRET,
    )(bg, cep, sl1, labels_t)
    return (o1[0, 0], o2[0, 0])


# speedup vs baseline: 2.6179x; 1.4128x over previous
import functools

import jax
import jax.numpy as jnp
from jax.experimental import pallas as pl

_NEG_POS_RATIO = 3
_INTERPRET = False


def _phase1_body(conf_ref, labels_ref, pred_ref, gt_ref, pos4_ref,
                 bg_ref, cep_ref, sl1_ref):
    x = conf_ref[0]
    xt = x.T
    m = jnp.max(xt, axis=0, keepdims=True)
    s = jnp.sum(jnp.exp(xt - m), axis=0, keepdims=True)
    lse = m + jnp.log(s)
    bg_ref[...] = (lse - xt[0:1, :])[None]
    lab = labels_ref[0]
    iota = jax.lax.broadcasted_iota(jnp.int32, xt.shape, 0)
    xsel = jnp.sum(jnp.where(iota == lab, xt, 0.0), axis=0, keepdims=True)
    cep = jnp.where(lab > 0, lse - xsel, 0.0)
    cep_ref[...] = jnp.sum(cep, axis=1, keepdims=True)[None]
    d = pred_ref[...] - gt_ref[...]
    ad = jnp.abs(d)
    sl1 = jnp.where(ad < 1.0, 0.5 * d * d, ad - 0.5)
    sl1 = jnp.where(pos4_ref[...], sl1, 0.0)
    sl1_ref[...] = jnp.sum(sl1, axis=2, keepdims=True)


def _phase2_body(neg_pos_ratio, n, bg_ref, ceps_ref, sl1s_ref, labels_ref,
                 o1_ref, o2_ref):
    lab = labels_ref[...]
    pos = lab > 0
    num_pos = jnp.sum(pos.astype(jnp.int32), axis=1, keepdims=True)
    k = num_pos * neg_pos_ratio
    negcount = n - num_pos
    bg = bg_ref[...]
    bits = jax.lax.bitcast_convert_type(bg, jnp.uint32)
    key = jnp.where(bg >= 0, bits | jnp.uint32(0x80000000), ~bits)
    key = jnp.where(pos, jnp.uint32(0), key)
    sum_neg = jnp.sum(jnp.where(pos, 0.0, bg), axis=1, keepdims=True)
    prefix = jnp.zeros_like(num_pos, dtype=jnp.uint32)
    kk = k
    for bit in range(31, -1, -1):
        cand = prefix | jnp.uint32(1 << bit)
        match = (key >> jnp.uint32(bit)) == (cand >> jnp.uint32(bit))
        cnt = jnp.sum(match.astype(jnp.int32), axis=1, keepdims=True)
        take = cnt >= kk
        prefix = jnp.where(take, cand, prefix)
        kk = jnp.where(take, kk, kk - cnt)
    t = prefix
    gt_mask = key > t
    num_gt = jnp.sum(gt_mask.astype(jnp.int32), axis=1, keepdims=True)
    sum_gt = jnp.sum(jnp.where(gt_mask, bg, 0.0), axis=1, keepdims=True)
    vt = jnp.max(jnp.where(key == t, bg, -jnp.inf), axis=1, keepdims=True)
    topk = sum_gt + (k - num_gt).astype(jnp.float32) * vt
    topk = jnp.where(k >= negcount, sum_neg, topk)
    topk = jnp.where(k <= 0, 0.0, topk)
    npos_tot = jnp.sum(num_pos, axis=0, keepdims=True).astype(jnp.float32) + 1e-6
    sl1_tot = jnp.sum(sl1s_ref[...], axis=0, keepdims=True)
    cls_tot = jnp.sum(ceps_ref[...] + topk, axis=0, keepdims=True)
    o1_ref[...] = sl1_tot / npos_tot
    o2_ref[...] = cls_tot / npos_tot


def kernel(confidence, predicted_locations, labels, gt_locations):
    b, n, c = confidence.shape
    labels = labels.astype(jnp.int32)
    labels3 = labels.reshape(b, 1, n)
    pos4 = jnp.broadcast_to((labels > 0)[:, :, None], (b, n, 4)).reshape(b, 1, 4 * n)
    pred2 = predicted_locations.reshape(b, 1, 4 * n)
    gt2 = gt_locations.reshape(b, 1, 4 * n)
    bg, ceps, sl1s = pl.pallas_call(
        _phase1_body,
        grid=(b,),
        in_specs=[
            pl.BlockSpec((1, n, c), lambda i: (i, 0, 0)),
            pl.BlockSpec((1, 1, n), lambda i: (i, 0, 0)),
            pl.BlockSpec((1, 1, 4 * n), lambda i: (i, 0, 0)),
            pl.BlockSpec((1, 1, 4 * n), lambda i: (i, 0, 0)),
            pl.BlockSpec((1, 1, 4 * n), lambda i: (i, 0, 0)),
        ],
        out_specs=[
            pl.BlockSpec((1, 1, n), lambda i: (i, 0, 0)),
            pl.BlockSpec((1, 1, 1), lambda i: (i, 0, 0)),
            pl.BlockSpec((1, 1, 1), lambda i: (i, 0, 0)),
        ],
        out_shape=[
            jax.ShapeDtypeStruct((b, 1, n), jnp.float32),
            jax.ShapeDtypeStruct((b, 1, 1), jnp.float32),
            jax.ShapeDtypeStruct((b, 1, 1), jnp.float32),
        ],
        interpret=_INTERPRET,
    )(confidence, labels3, pred2, gt2, pos4)

    o1, o2 = pl.pallas_call(
        functools.partial(_phase2_body, _NEG_POS_RATIO, n),
        out_shape=[
            jax.ShapeDtypeStruct((1, 1), jnp.float32),
            jax.ShapeDtypeStruct((1, 1), jnp.float32),
        ],
        interpret=_INTERPRET,
    )(bg.reshape(b, n), ceps.reshape(b, 1), sl1s.reshape(b, 1), labels)
    return (o1[0, 0], o2[0, 0])


# speedup vs baseline: 11.3972x; 4.3535x over previous
import functools

import jax
import jax.numpy as jnp
from jax.experimental import pallas as pl
from jax.experimental.pallas import tpu as pltpu

_NEG_POS_RATIO = 3
_INTERPRET = False


def _lse_body(conf_ref, labels_ref, bg_ref, cep_ref, m_s, s_s, xsel_s, x0_s):
    nc = pl.num_programs(1)
    kc = pl.program_id(1)
    x = conf_ref[...]
    cc = x.shape[0]
    lab = labels_ref[...]
    mx = jnp.max(x, axis=0)
    iota = jax.lax.broadcasted_iota(jnp.int32, x.shape, 0) + kc * cc
    xsel_c = jnp.sum(jnp.where(iota == lab[None], x, 0.0), axis=0)

    @pl.when(kc == 0)
    def _init():
        m_s[...] = mx
        s_s[...] = jnp.sum(jnp.exp(x - mx[None]), axis=0)
        xsel_s[...] = xsel_c
        x0_s[...] = x[0]

    @pl.when(kc > 0)
    def _update():
        m_old = m_s[...]
        m_new = jnp.maximum(m_old, mx)
        s_s[...] = (s_s[...] * jnp.exp(m_old - m_new)
                    + jnp.sum(jnp.exp(x - m_new[None]), axis=0))
        m_s[...] = m_new
        xsel_s[...] = xsel_s[...] + xsel_c

    @pl.when(kc == nc - 1)
    def _emit():
        lse = m_s[...] + jnp.log(s_s[...])
        bg_ref[...] = lse - x0_s[...]
        cep = jnp.where(lab > 0, lse - xsel_s[...], 0.0)
        cep_ref[...] = jnp.sum(cep, axis=1, keepdims=True)


def _sl1_body(pred_ref, gt_ref, labels_ref, sl1_ref):
    d = pred_ref[...] - gt_ref[...]
    ad = jnp.abs(d)
    sl1 = jnp.where(ad < 1.0, 0.5 * d * d, ad - 0.5)
    s = jnp.sum(sl1, axis=1)
    masked = jnp.where(labels_ref[...] > 0, s, 0.0)
    sl1_ref[...] = jnp.sum(masked, axis=1, keepdims=True)


def _mine_body(neg_pos_ratio, n, bg_ref, ceps_ref, sl1s_ref, labels_ref,
               o1_ref, o2_ref):
    lab = labels_ref[...]
    pos = lab > 0
    num_pos = jnp.sum(pos.astype(jnp.int32), axis=1, keepdims=True)
    k = num_pos * neg_pos_ratio
    negcount = n - num_pos
    bg = bg_ref[...]
    bits = jax.lax.bitcast_convert_type(bg, jnp.uint32)
    key = jnp.where(bg >= 0, bits | jnp.uint32(0x80000000), ~bits)
    key = jnp.where(pos, jnp.uint32(0), key)
    sum_neg = jnp.sum(jnp.where(pos, 0.0, bg), axis=1, keepdims=True)
    prefix = jnp.zeros_like(num_pos, dtype=jnp.uint32)
    kk = k
    for bit in range(31, -1, -1):
        cand = prefix | jnp.uint32(1 << bit)
        match = (key >> jnp.uint32(bit)) == (cand >> jnp.uint32(bit))
        cnt = jnp.sum(match.astype(jnp.int32), axis=1, keepdims=True)
        take = cnt >= kk
        prefix = jnp.where(take, cand, prefix)
        kk = jnp.where(take, kk, kk - cnt)
    t = prefix
    gt_mask = key > t
    num_gt = jnp.sum(gt_mask.astype(jnp.int32), axis=1, keepdims=True)
    sum_gt = jnp.sum(jnp.where(gt_mask, bg, 0.0), axis=1, keepdims=True)
    vt = jnp.max(jnp.where(key == t, bg, -jnp.inf), axis=1, keepdims=True)
    topk = sum_gt + (k - num_gt).astype(jnp.float32) * vt
    topk = jnp.where(k >= negcount, sum_neg, topk)
    topk = jnp.where(k <= 0, 0.0, topk)
    npos_tot = jnp.sum(num_pos, axis=0, keepdims=True).astype(jnp.float32) + 1e-6
    sl1_tot = jnp.sum(sl1s_ref[...], axis=0, keepdims=True)
    cls_tot = jnp.sum(ceps_ref[...] + topk, axis=0, keepdims=True)
    o1_ref[...] = sl1_tot / npos_tot
    o2_ref[...] = cls_tot / npos_tot


def kernel(confidence, predicted_locations, labels, gt_locations):
    b, n, c = confidence.shape
    labels = labels.astype(jnp.int32)
    conf_t = confidence.transpose(2, 0, 1)
    pred_t = predicted_locations.transpose(0, 2, 1)
    gt_t = gt_locations.transpose(0, 2, 1)
    bb = 8
    cc = 9
    bg, ceps = pl.pallas_call(
        _lse_body,
        grid=(b // bb, c // cc),
        in_specs=[
            pl.BlockSpec((cc, bb, n), lambda j, kc: (kc, j, 0)),
            pl.BlockSpec((bb, n), lambda j, kc: (j, 0)),
        ],
        out_specs=[
            pl.BlockSpec((bb, n), lambda j, kc: (j, 0)),
            pl.BlockSpec((bb, 1), lambda j, kc: (j, 0)),
        ],
        out_shape=[
            jax.ShapeDtypeStruct((b, n), jnp.float32),
            jax.ShapeDtypeStruct((b, 1), jnp.float32),
        ],
        scratch_shapes=[pltpu.VMEM((bb, n), jnp.float32) for _ in range(4)],
        interpret=_INTERPRET,
    )(conf_t, labels)

    sl1s = pl.pallas_call(
        _sl1_body,
        grid=(b // bb,),
        in_specs=[
            pl.BlockSpec((bb, 4, n), lambda j: (j, 0, 0)),
            pl.BlockSpec((bb, 4, n), lambda j: (j, 0, 0)),
            pl.BlockSpec((bb, n), lambda j: (j, 0)),
        ],
        out_specs=pl.BlockSpec((bb, 1), lambda j: (j, 0)),
        out_shape=jax.ShapeDtypeStruct((b, 1), jnp.float32),
        interpret=_INTERPRET,
    )(pred_t, gt_t, labels)

    o1, o2 = pl.pallas_call(
        functools.partial(_mine_body, _NEG_POS_RATIO, n),
        out_shape=[
            jax.ShapeDtypeStruct((1, 1), jnp.float32),
            jax.ShapeDtypeStruct((1, 1), jnp.float32),
        ],
        interpret=_INTERPRET,
    )(bg, ceps, sl1s, labels)
    return (o1[0, 0], o2[0, 0])
